# Initial kernel scaffold; baseline (speedup 1.0000x reference)
import jax
import jax.numpy as jnp
from jax.experimental import pallas as pl


def kernel(input, target):
    raise NotImplementedError("write your pallas kernel here")


# speedup vs baseline: 137.0179x; 137.0179x over previous
import functools

import jax
import jax.numpy as jnp
from jax.experimental import pallas as pl
from jax.experimental.pallas import tpu as pltpu

_B = 128
_C = 100000
_W = 8192


def _loss_kernel(x_ref, t_ref, o_ref, sum_acc, tgt_acc, *, nblk, width, ncols):
    j = pl.program_id(0)
    x = x_ref[...]
    b, w = x.shape
    col = j * width + jax.lax.broadcasted_iota(jnp.int32, (b, w), 1)
    e = jnp.where(col < ncols, jnp.exp(x), 0.0)
    s = jnp.sum(e, axis=1, keepdims=True)
    tv = jnp.sum(jnp.where(col == t_ref[...], x, 0.0), axis=1, keepdims=True)

    @pl.when(j == 0)
    def _init():
        sum_acc[...] = s
        tgt_acc[...] = tv

    @pl.when(j > 0)
    def _accum():
        sum_acc[...] += s
        tgt_acc[...] += tv

    @pl.when(j == nblk - 1)
    def _finish():
        p = jnp.exp(tgt_acc[...]) / sum_acc[...]
        o_ref[...] = -jnp.mean(jnp.log(p + 1e-6)).reshape(1, 1)


def kernel(input, target):
    b, c = input.shape
    nblk = pl.cdiv(c, _W)
    t2 = target.astype(jnp.int32).reshape(b, 1)
    out = pl.pallas_call(
        functools.partial(_loss_kernel, nblk=nblk, width=_W, ncols=c),
        grid=(nblk,),
        in_specs=[
            pl.BlockSpec((b, _W), lambda j: (0, j)),
            pl.BlockSpec((b, 1), lambda j: (0, 0)),
        ],
        out_specs=pl.BlockSpec((1, 1), lambda j: (0, 0)),
        out_shape=jax.ShapeDtypeStruct((1, 1), jnp.float32),
        scratch_shapes=[
            pltpu.VMEM((b, 1), jnp.float32),
            pltpu.VMEM((b, 1), jnp.float32),
        ],
    )(input, t2)
    return out[0, 0]


# speedup vs baseline: 138.8545x; 1.0134x over previous
import functools

import jax
import jax.numpy as jnp
from jax.experimental import pallas as pl
from jax.experimental.pallas import tpu as pltpu

_B = 128
_C = 100000
_W = 16384


def _loss_kernel(x_ref, t_ref, o_ref, sum_acc, tgt_acc, *, nblk, width, ncols):
    j = pl.program_id(0)
    x = x_ref[...]
    b, w = x.shape
    col = j * width + jax.lax.broadcasted_iota(jnp.int32, (b, w), 1)
    e = jnp.where(col < ncols, jnp.exp(x), 0.0)
    s = jnp.sum(e, axis=1, keepdims=True)
    tv = jnp.sum(jnp.where(col == t_ref[...], x, 0.0), axis=1, keepdims=True)

    @pl.when(j == 0)
    def _init():
        sum_acc[...] = s
        tgt_acc[...] = tv

    @pl.when(j > 0)
    def _accum():
        sum_acc[...] += s
        tgt_acc[...] += tv

    @pl.when(j == nblk - 1)
    def _finish():
        p = jnp.exp(tgt_acc[...]) / sum_acc[...]
        o_ref[...] = -jnp.mean(jnp.log(p + 1e-6)).reshape(1, 1)


def kernel(input, target):
    b, c = input.shape
    nblk = pl.cdiv(c, _W)
    t2 = target.astype(jnp.int32).reshape(b, 1)
    out = pl.pallas_call(
        functools.partial(_loss_kernel, nblk=nblk, width=_W, ncols=c),
        grid=(nblk,),
        in_specs=[
            pl.BlockSpec((b, _W), lambda j: (0, j)),
            pl.BlockSpec((b, 1), lambda j: (0, 0)),
        ],
        out_specs=pl.BlockSpec((1, 1), lambda j: (0, 0)),
        out_shape=jax.ShapeDtypeStruct((1, 1), jnp.float32),
        scratch_shapes=[
            pltpu.VMEM((b, 1), jnp.float32),
            pltpu.VMEM((b, 1), jnp.float32),
        ],
    )(input, t2)
    return out[0, 0]
